# baseline (speedup 1.0000x reference)
import functools
import math

import jax
import jax.numpy as jnp
import numpy as np
from jax import lax
from jax.experimental import pallas as pl
from jax.experimental.pallas import tpu as pltpu
from jax.experimental.pallas import tpu_sc as plsc

_NUM_LEVELS = 16
_LEVEL_DIM = 2
_BASE_RES = 16
_T = 2 ** 19
_DESIRED = 512
_N = 262144
_P1 = np.int32(-1640531535)
_P2 = np.int32(805459861)
_MASK = _T - 1

_PLS = math.exp(math.log(_DESIRED / _BASE_RES) / (_NUM_LEVELS - 1))
_RES = [int(np.floor(_BASE_RES * _PLS ** l)) for l in range(_NUM_LEVELS)]

_NC, _NS, _L = 2, 16, 16
_NW = _NC * _NS
_PTS_PER_W = _N // _NW
_C = 512
_CHUNKS = _PTS_PER_W // _C
_VPC = _C // _L
_ROWS = 8 * _C

_LPG = 2
_NG = _NUM_LEVELS // _LPG
_SPM = _LPG * _T
_STG = _SPM // _NS


@functools.lru_cache(maxsize=None)
def _build_hash_encode_sc(n):
    mesh = plsc.VectorSubcoreMesh(
        core_axis_name="c", subcore_axis_name="s",
        num_cores=_NC, num_subcores=_NS,
    )
    ppw = n // _NW
    return pl.kernel(
        functools.partial(_hash_encode_body, ppw),
        out_type=jax.ShapeDtypeStruct((2 * _NUM_LEVELS, n), jnp.float32),
        mesh=mesh,
        compiler_params=pltpu.CompilerParams(needs_layout_passes=False),
        scratch_types=[
            pltpu.VMEM((3, ppw), jnp.float32),
            pltpu.VMEM((_ROWS,), jnp.int32),
            pltpu.VMEM((_ROWS,), jnp.int32),
            pltpu.VMEM((_ROWS,), jnp.int32),
            pltpu.VMEM((_ROWS,), jnp.int32),
            pltpu.VMEM((2 * _LPG, _C), jnp.float32),
            pltpu.VMEM_SHARED((_SPM,), jnp.int32),
            pltpu.SemaphoreType.DMA,
            pltpu.SemaphoreType.DMA,
        ],
    )


def _hash_encode_body(ppw, xT, tab, out, xs_ref, idx_a, idx_b, rows_a, rows_b,
                      outb_a, spm, sem_a, sem_b):
    wid = lax.axis_index("s") * _NC + lax.axis_index("c")
    sid = lax.axis_index("s")
    base = wid * ppw
    n_chunks = ppw // _C
    idx_refs = (idx_a, idx_b)
    rows_refs = (rows_a, rows_b)
    sems = (sem_a, sem_b)

    pltpu.sync_copy(xT.at[:, pl.ds(base, ppw)], xs_ref)

    def phase1(l, ci, idx_ref):
        res = float(_RES[l])
        lofs = (l % _LPG) * _T

        def body(j, carry):
            o = ci * _C + j * _L
            x0 = xs_ref[0, pl.ds(o, _L)]
            x1 = xs_ref[1, pl.ds(o, _L)]
            x2 = xs_ref[2, pl.ds(o, _L)]

            def cell(xd):
                x01 = jnp.minimum(jnp.maximum((xd + 1.0) * 0.5, 0.0), 1.0)
                return (x01 * res).astype(jnp.int32)

            p0 = cell(x0)
            p1 = cell(x1)
            p2 = cell(x2)
            t0 = (p0, p0 + 1)
            t1 = (p1 * _P1, (p1 + 1) * _P1)
            t2 = (p2 * _P2, (p2 + 1) * _P2)
            for k in range(8):
                b0, b1, b2 = k & 1, (k >> 1) & 1, (k >> 2) & 1
                h = (t0[b0] ^ t1[b1] ^ t2[b2]) & _MASK
                idx_ref[pl.ds(k * _C + j * _L, _L)] = h + lofs
            return carry

        lax.fori_loop(0, _VPC, body, 0)

    def phase2(l, ci, rows_ref, outb):
        res = float(_RES[l])

        def body(j, carry):
            o = ci * _C + j * _L
            x0 = xs_ref[0, pl.ds(o, _L)]
            x1 = xs_ref[1, pl.ds(o, _L)]
            x2 = xs_ref[2, pl.ds(o, _L)]

            def frac(xd):
                x01 = jnp.minimum(jnp.maximum((xd + 1.0) * 0.5, 0.0), 1.0)
                pos = x01 * res
                return pos - pos.astype(jnp.int32).astype(jnp.float32)

            f0 = frac(x0)
            f1 = frac(x1)
            f2 = frac(x2)
            w0 = (1.0 - f0, f0)
            w1 = (1.0 - f1, f1)
            w2 = (1.0 - f2, f2)
            acc0 = jnp.zeros((16,), jnp.float32)
            acc1 = jnp.zeros((16,), jnp.float32)
            for k in range(8):
                b0, b1, b2 = k & 1, (k >> 1) & 1, (k >> 2) & 1
                w = w0[b0] * w1[b1] * w2[b2]
                pair = rows_ref[pl.ds(k * _C + j * _L, _L)]
                v0 = plsc.bitcast(lax.shift_left(pair, 16), jnp.float32)
                v1 = plsc.bitcast(pair & np.int32(-65536), jnp.float32)
                acc0 = acc0 + w * v0
                acc1 = acc1 + w * v1
            lrow = 2 * (l % _LPG)
            outb[lrow, pl.ds(j * _L, _L)] = acc0
            outb[lrow + 1, pl.ds(j * _L, _L)] = acc1
            return carry

        lax.fori_loop(0, _VPC, body, 0)

    for g in range(_NG):
        plsc.subcore_barrier()
        pltpu.sync_copy(tab.at[pl.ds(g * _SPM + sid * _STG, _STG)],
                        spm.at[pl.ds(sid * _STG, _STG)])
        plsc.subcore_barrier()
        l0 = g * _LPG
        l1 = l0 + 1

        def chunk_body(ci, carry, l0=l0, l1=l1, g=g):
            phase1(l0, ci, idx_refs[0])
            cp0 = pltpu.async_copy(spm.at[idx_refs[0]], rows_refs[0], sems[0])
            phase1(l1, ci, idx_refs[1])
            cp1 = pltpu.async_copy(spm.at[idx_refs[1]], rows_refs[1], sems[1])
            cp0.wait()
            phase2(l0, ci, rows_refs[0], outb_a)
            cp1.wait()
            phase2(l1, ci, rows_refs[1], outb_a)
            pltpu.sync_copy(
                outb_a,
                out.at[pl.ds(4 * g, 2 * _LPG), pl.ds(base + ci * _C, _C)])
            return carry

        lax.fori_loop(0, n_chunks, chunk_body, 0)


_PAIRS = _NUM_LEVELS * _T
_PPW = _PAIRS // _NW
_PCH = 8192


@functools.lru_cache(maxsize=None)
def _build_pack_sc():
    mesh = plsc.VectorSubcoreMesh(
        core_axis_name="c", subcore_axis_name="s",
        num_cores=_NC, num_subcores=_NS,
    )
    return pl.kernel(
        _pack_sc_body,
        out_type=jax.ShapeDtypeStruct((_PAIRS,), jnp.int32),
        mesh=mesh,
        compiler_params=pltpu.CompilerParams(needs_layout_passes=False),
        scratch_types=[
            pltpu.VMEM((2 * _PCH,), jnp.float32),
            pltpu.VMEM((2 * _PCH,), jnp.float32),
            pltpu.VMEM((_PCH,), jnp.int32),
            pltpu.VMEM((_PCH,), jnp.int32),
            pltpu.SemaphoreType.DMA,
            pltpu.SemaphoreType.DMA,
            pltpu.SemaphoreType.DMA,
            pltpu.SemaphoreType.DMA,
        ],
    )


def _pack_sc_body(tab, out, in_a, in_b, out_a, out_b,
                  isem_a, isem_b, osem_a, osem_b):
    wid = lax.axis_index("s") * _NC + lax.axis_index("c")
    ebase = wid * (2 * _PPW)
    pbase = wid * _PPW
    ins = (in_a, in_b)
    outs = (out_a, out_b)
    isems = (isem_a, isem_b)
    osems = (osem_a, osem_b)
    nch = _PPW // _PCH

    def rnd(v):
        u = plsc.bitcast(v, jnp.int32)
        return lax.shift_right_logical(u + (32767 + ((u >> 16) & 1)), 16)

    def fire_in(ci):
        pb = ci & 1
        return pltpu.async_copy(
            tab.at[pl.ds(ebase + ci * (2 * _PCH), 2 * _PCH)], ins[pb],
            isems[pb])

    in_cps = {0: fire_in(0)}
    out_cps = {}
    for ci in range(nch):
        pb = ci & 1
        if ci + 1 < nch:
            in_cps[ci + 1] = fire_in(ci + 1)
        in_cps.pop(ci).wait()
        if ci - 2 in out_cps:
            out_cps.pop(ci - 2).wait()
        in_ref = ins[pb]
        out_ref = outs[pb]

        def vb_body(vb, c2, in_ref=in_ref, out_ref=out_ref):
            s = lax.shift_right_logical(vb, 3) * 256 + (vb & 7) * _L
            r0 = rnd(in_ref[pl.ds(s, _L)])
            r1 = rnd(in_ref[pl.ds(s + 128, _L)])
            out_ref[pl.ds(vb * _L, _L)] = r0 | lax.shift_left(r1, 16)
            return c2

        lax.fori_loop(0, _PCH // _L, vb_body, 0)
        out_cps[ci] = pltpu.async_copy(
            out_ref, out.at[pl.ds(pbase + ci * _PCH, _PCH)], osems[pb])
    for cp in out_cps.values():
        cp.wait()


_B = 1024


def _mlp_body(encT_ref, dT_ref, sw0T_ref, sw1T_ref, sw2T_ref, cw0T_ref,
              cw1T_ref, out_ref):
    dot = functools.partial(jnp.dot, precision=lax.Precision.DEFAULT,
                            preferred_element_type=jnp.float32)
    e = encT_ref[...]
    h = jnp.maximum(dot(sw0T_ref[...], e), 0.0)
    h = jnp.maximum(dot(sw1T_ref[...], h), 0.0)
    h2 = dot(sw2T_ref[...], h)
    geo = h2[0:64, :]
    dx = dT_ref[0, :]
    dy = dT_ref[1, :]
    dz = dT_ref[2, :]
    xx, yy, zz = dx * dx, dy * dy, dz * dz
    xy, yz, xz = dx * dy, dy * dz, dx * dz
    sh_rows = [
        0.28209479177387814 * jnp.ones_like(dx),
        -0.48860251190291987 * dy,
        0.48860251190291987 * dz,
        -0.48860251190291987 * dx,
        1.0925484305920792 * xy,
        -1.0925484305920792 * yz,
        0.94617469575755997 * zz - 0.31539156525252005,
        -1.0925484305920792 * xz,
        0.54627421529603959 * (xx - yy),
        0.59004358992664352 * dy * (-3.0 * xx + yy),
        2.8906114426405538 * xy * dz,
        0.45704579946446572 * dy * (1.0 - 5.0 * zz),
        0.3731763325901154 * dz * (5.0 * zz - 3.0),
        0.45704579946446572 * dx * (1.0 - 5.0 * zz),
        1.445305721320277 * dz * (xx - yy),
        0.59004358992664352 * dx * (-xx + 3.0 * yy),
    ]
    sh = jnp.concatenate([r[None, :] for r in sh_rows], axis=0)
    cat = jnp.concatenate([sh, geo], axis=0)
    c = jnp.maximum(dot(cw0T_ref[...], cat), 0.0)
    o = dot(cw1T_ref[...], c)
    col = jax.nn.sigmoid(o) * 1.002 - 0.001
    rid = lax.broadcasted_iota(jnp.int32, (8, _B), 0)
    out_ref[...] = jnp.where(rid == 3,
                             jnp.broadcast_to(h2[64:65, :], (8, _B)), col)


@functools.lru_cache(maxsize=None)
def _build_mlp_call(n):
    return pl.pallas_call(
        _mlp_body,
        grid=(n // _B,),
        in_specs=[
            pl.BlockSpec((2 * _NUM_LEVELS, _B), lambda i: (0, i)),
            pl.BlockSpec((8, _B), lambda i: (0, i)),
            pl.BlockSpec((64, 32), lambda i: (0, 0)),
            pl.BlockSpec((64, 64), lambda i: (0, 0)),
            pl.BlockSpec((72, 64), lambda i: (0, 0)),
            pl.BlockSpec((64, 80), lambda i: (0, 0)),
            pl.BlockSpec((8, 64), lambda i: (0, 0)),
        ],
        out_specs=pl.BlockSpec((8, _B), lambda i: (0, i)),
        out_shape=jax.ShapeDtypeStruct((8, n), jnp.float32),
    )


def kernel(x, d, table, sw0, sw1, sw2, cw0, cw1):
    xT = x.T
    tab = jnp.transpose(
        jnp.transpose(table, (0, 2, 1)).reshape(_NUM_LEVELS, _LEVEL_DIM,
                                                _T // 128, 128),
        (0, 2, 1, 3),
    ).reshape(_NUM_LEVELS * _T * _LEVEL_DIM)
    packed = _build_pack_sc()(tab)
    dT8 = jnp.concatenate([d.T, jnp.zeros((5, _N), jnp.float32)], axis=0)
    sw0T = sw0.T
    sw1T = sw1.T
    sw2Tp = jnp.concatenate([sw2[:, 1:], sw2[:, :1]], axis=1).T
    sw2Tp = jnp.concatenate([sw2Tp, jnp.zeros((7, 64), jnp.float32)], axis=0)
    cw0T = cw0.T
    cw1T8 = jnp.concatenate([cw1.T, jnp.zeros((5, 64), jnp.float32)], axis=0)
    splits = (0, 3 * _N // 4, _N)
    encs = []
    for i in range(2):
        lo, hi = splits[i], splits[i + 1]
        encs.append(_build_hash_encode_sc(hi - lo)(
            lax.slice(xT, (0, lo), (3, hi)), packed))
    outs = []
    for i in range(2):
        lo, hi = splits[i], splits[i + 1]
        outs.append(_build_mlp_call(hi - lo)(
            encs[i], lax.slice(dT8, (0, lo), (8, hi)),
            sw0T, sw1T, sw2Tp, cw0T, cw1T8))
    out8 = jnp.concatenate(outs, axis=1)
    sigma = out8[3]
    color = out8[:3].T
    return (sigma, color)


# speedup vs baseline: 1.1563x; 1.1563x over previous
import functools
import math

import jax
import jax.numpy as jnp
import numpy as np
from jax import lax
from jax.experimental import pallas as pl
from jax.experimental.pallas import tpu as pltpu
from jax.experimental.pallas import tpu_sc as plsc

_NUM_LEVELS = 16
_LEVEL_DIM = 2
_BASE_RES = 16
_T = 2 ** 19
_DESIRED = 512
_N = 262144
_P1 = np.int32(-1640531535)
_P2 = np.int32(805459861)
_MASK = _T - 1

_PLS = math.exp(math.log(_DESIRED / _BASE_RES) / (_NUM_LEVELS - 1))
_RES = [int(np.floor(_BASE_RES * _PLS ** l)) for l in range(_NUM_LEVELS)]

_NC, _NS, _L = 2, 16, 16
_NW = _NC * _NS
_PTS_PER_W = _N // _NW
_C = 512
_CHUNKS = _PTS_PER_W // _C
_VPC = _C // _L
_ROWS = 8 * _C

_LPG = 2
_NG = _NUM_LEVELS // _LPG
_SPM = _LPG * _T
_STG = _SPM // _NS


@functools.lru_cache(maxsize=None)
def _build_hash_encode_sc(n):
    mesh = plsc.VectorSubcoreMesh(
        core_axis_name="c", subcore_axis_name="s",
        num_cores=_NC, num_subcores=_NS,
    )
    ppw = n // _NW
    return pl.kernel(
        functools.partial(_hash_encode_body, ppw),
        out_type=jax.ShapeDtypeStruct((2 * _NUM_LEVELS, n), jnp.float32),
        mesh=mesh,
        compiler_params=pltpu.CompilerParams(needs_layout_passes=False),
        scratch_types=[
            pltpu.VMEM((3, ppw), jnp.float32),
            pltpu.VMEM((_ROWS,), jnp.int32),
            pltpu.VMEM((_ROWS,), jnp.int32),
            pltpu.VMEM((_ROWS,), jnp.int32),
            pltpu.VMEM((_ROWS,), jnp.int32),
            pltpu.VMEM((2 * _LPG, _C), jnp.float32),
            pltpu.VMEM_SHARED((_SPM,), jnp.int32),
            pltpu.SemaphoreType.DMA,
            pltpu.SemaphoreType.DMA,
        ],
    )


def _hash_encode_body(ppw, xT, tab, out, xs_ref, idx_a, idx_b, rows_a, rows_b,
                      outb_a, spm, sem_a, sem_b):
    wid = lax.axis_index("s") * _NC + lax.axis_index("c")
    sid = lax.axis_index("s")
    base = wid * ppw
    n_chunks = ppw // _C
    idx_refs = (idx_a, idx_b)
    rows_refs = (rows_a, rows_b)
    sems = (sem_a, sem_b)

    pltpu.sync_copy(xT.at[:, pl.ds(base, ppw)], xs_ref)

    def phase1(l, ci, idx_ref):
        res = float(_RES[l])
        lofs = (l % _LPG) * _T

        def body(j, carry):
            o = ci * _C + j * _L
            x0 = xs_ref[0, pl.ds(o, _L)]
            x1 = xs_ref[1, pl.ds(o, _L)]
            x2 = xs_ref[2, pl.ds(o, _L)]

            def cell(xd):
                x01 = jnp.minimum(jnp.maximum((xd + 1.0) * 0.5, 0.0), 1.0)
                return (x01 * res).astype(jnp.int32)

            p0 = cell(x0)
            p1 = cell(x1)
            p2 = cell(x2)
            t0 = (p0, p0 + 1)
            t1 = (p1 * _P1, (p1 + 1) * _P1)
            t2 = (p2 * _P2, (p2 + 1) * _P2)
            for k in range(8):
                b0, b1, b2 = k & 1, (k >> 1) & 1, (k >> 2) & 1
                h = (t0[b0] ^ t1[b1] ^ t2[b2]) & _MASK
                idx_ref[pl.ds(k * _C + j * _L, _L)] = h + lofs
            return carry

        lax.fori_loop(0, _VPC, body, 0)

    def phase2(l, ci, rows_ref, outb):
        res = float(_RES[l])

        def body(j, carry):
            o = ci * _C + j * _L
            x0 = xs_ref[0, pl.ds(o, _L)]
            x1 = xs_ref[1, pl.ds(o, _L)]
            x2 = xs_ref[2, pl.ds(o, _L)]

            def frac(xd):
                x01 = jnp.minimum(jnp.maximum((xd + 1.0) * 0.5, 0.0), 1.0)
                pos = x01 * res
                return pos - pos.astype(jnp.int32).astype(jnp.float32)

            f0 = frac(x0)
            f1 = frac(x1)
            f2 = frac(x2)
            w0 = (1.0 - f0, f0)
            w1 = (1.0 - f1, f1)
            w2 = (1.0 - f2, f2)
            acc0 = jnp.zeros((16,), jnp.float32)
            acc1 = jnp.zeros((16,), jnp.float32)
            for k in range(8):
                b0, b1, b2 = k & 1, (k >> 1) & 1, (k >> 2) & 1
                w = w0[b0] * w1[b1] * w2[b2]
                pair = rows_ref[pl.ds(k * _C + j * _L, _L)]
                v0 = plsc.bitcast(lax.shift_left(pair, 16), jnp.float32)
                v1 = plsc.bitcast(pair & np.int32(-65536), jnp.float32)
                acc0 = acc0 + w * v0
                acc1 = acc1 + w * v1
            lrow = 2 * (l % _LPG)
            outb[lrow, pl.ds(j * _L, _L)] = acc0
            outb[lrow + 1, pl.ds(j * _L, _L)] = acc1
            return carry

        lax.fori_loop(0, _VPC, body, 0)

    for g in range(_NG):
        plsc.subcore_barrier()
        pltpu.sync_copy(tab.at[pl.ds(g * _SPM + sid * _STG, _STG)],
                        spm.at[pl.ds(sid * _STG, _STG)])
        plsc.subcore_barrier()
        l0 = g * _LPG
        l1 = l0 + 1

        phase1(l0, 0, idx_refs[0])
        pltpu.async_copy(spm.at[idx_refs[0]], rows_refs[0], sems[0])
        phase1(l1, 0, idx_refs[1])
        pltpu.async_copy(spm.at[idx_refs[1]], rows_refs[1], sems[1])

        def chunk_body(ci, carry, l0=l0, l1=l1, g=g):
            for b, l in ((0, l0), (1, l1)):
                pltpu.make_async_copy(spm.at[idx_refs[b]], rows_refs[b],
                                      sems[b]).wait()
                phase2(l, ci, rows_refs[b], outb_a)

                @pl.when(ci + 1 < n_chunks)
                def _(b=b, l=l):
                    phase1(l, ci + 1, idx_refs[b])
                    pltpu.async_copy(spm.at[idx_refs[b]], rows_refs[b],
                                     sems[b])

            pltpu.sync_copy(
                outb_a,
                out.at[pl.ds(4 * g, 2 * _LPG), pl.ds(base + ci * _C, _C)])
            return carry

        lax.fori_loop(0, n_chunks, chunk_body, 0)


_PAIRS = _NUM_LEVELS * _T
_PPW = _PAIRS // _NW
_PCH = 8192


@functools.lru_cache(maxsize=None)
def _build_pack_sc():
    mesh = plsc.VectorSubcoreMesh(
        core_axis_name="c", subcore_axis_name="s",
        num_cores=_NC, num_subcores=_NS,
    )
    return pl.kernel(
        _pack_sc_body,
        out_type=jax.ShapeDtypeStruct((_PAIRS,), jnp.int32),
        mesh=mesh,
        compiler_params=pltpu.CompilerParams(needs_layout_passes=False),
        scratch_types=[
            pltpu.VMEM((2 * _PCH,), jnp.float32),
            pltpu.VMEM((2 * _PCH,), jnp.float32),
            pltpu.VMEM((_PCH,), jnp.int32),
            pltpu.VMEM((_PCH,), jnp.int32),
            pltpu.SemaphoreType.DMA,
            pltpu.SemaphoreType.DMA,
            pltpu.SemaphoreType.DMA,
            pltpu.SemaphoreType.DMA,
        ],
    )


def _pack_sc_body(tab, out, in_a, in_b, out_a, out_b,
                  isem_a, isem_b, osem_a, osem_b):
    wid = lax.axis_index("s") * _NC + lax.axis_index("c")
    ebase = wid * (2 * _PPW)
    pbase = wid * _PPW
    ins = (in_a, in_b)
    outs = (out_a, out_b)
    isems = (isem_a, isem_b)
    osems = (osem_a, osem_b)
    nch = _PPW // _PCH

    def rnd(v):
        u = plsc.bitcast(v, jnp.int32)
        return lax.shift_right_logical(u + (32767 + ((u >> 16) & 1)), 16)

    def fire_in(ci):
        pb = ci & 1
        return pltpu.async_copy(
            tab.at[pl.ds(ebase + ci * (2 * _PCH), 2 * _PCH)], ins[pb],
            isems[pb])

    in_cps = {0: fire_in(0)}
    out_cps = {}
    for ci in range(nch):
        pb = ci & 1
        if ci + 1 < nch:
            in_cps[ci + 1] = fire_in(ci + 1)
        in_cps.pop(ci).wait()
        if ci - 2 in out_cps:
            out_cps.pop(ci - 2).wait()
        in_ref = ins[pb]
        out_ref = outs[pb]

        def vb_body(vb, c2, in_ref=in_ref, out_ref=out_ref):
            s = lax.shift_right_logical(vb, 3) * 256 + (vb & 7) * _L
            r0 = rnd(in_ref[pl.ds(s, _L)])
            r1 = rnd(in_ref[pl.ds(s + 128, _L)])
            out_ref[pl.ds(vb * _L, _L)] = r0 | lax.shift_left(r1, 16)
            return c2

        lax.fori_loop(0, _PCH // _L, vb_body, 0)
        out_cps[ci] = pltpu.async_copy(
            out_ref, out.at[pl.ds(pbase + ci * _PCH, _PCH)], osems[pb])
    for cp in out_cps.values():
        cp.wait()


_B = 1024


def _mlp_body(encT_ref, dT_ref, sw0T_ref, sw1T_ref, sw2T_ref, cw0T_ref,
              cw1T_ref, out_ref):
    dot = functools.partial(jnp.dot, precision=lax.Precision.DEFAULT,
                            preferred_element_type=jnp.float32)
    e = encT_ref[...]
    h = jnp.maximum(dot(sw0T_ref[...], e), 0.0)
    h = jnp.maximum(dot(sw1T_ref[...], h), 0.0)
    h2 = dot(sw2T_ref[...], h)
    geo = h2[0:64, :]
    dx = dT_ref[0, :]
    dy = dT_ref[1, :]
    dz = dT_ref[2, :]
    xx, yy, zz = dx * dx, dy * dy, dz * dz
    xy, yz, xz = dx * dy, dy * dz, dx * dz
    sh_rows = [
        0.28209479177387814 * jnp.ones_like(dx),
        -0.48860251190291987 * dy,
        0.48860251190291987 * dz,
        -0.48860251190291987 * dx,
        1.0925484305920792 * xy,
        -1.0925484305920792 * yz,
        0.94617469575755997 * zz - 0.31539156525252005,
        -1.0925484305920792 * xz,
        0.54627421529603959 * (xx - yy),
        0.59004358992664352 * dy * (-3.0 * xx + yy),
        2.8906114426405538 * xy * dz,
        0.45704579946446572 * dy * (1.0 - 5.0 * zz),
        0.3731763325901154 * dz * (5.0 * zz - 3.0),
        0.45704579946446572 * dx * (1.0 - 5.0 * zz),
        1.445305721320277 * dz * (xx - yy),
        0.59004358992664352 * dx * (-xx + 3.0 * yy),
    ]
    sh = jnp.concatenate([r[None, :] for r in sh_rows], axis=0)
    cat = jnp.concatenate([sh, geo], axis=0)
    c = jnp.maximum(dot(cw0T_ref[...], cat), 0.0)
    o = dot(cw1T_ref[...], c)
    col = jax.nn.sigmoid(o) * 1.002 - 0.001
    rid = lax.broadcasted_iota(jnp.int32, (8, _B), 0)
    out_ref[...] = jnp.where(rid == 3,
                             jnp.broadcast_to(h2[64:65, :], (8, _B)), col)


@functools.lru_cache(maxsize=None)
def _build_mlp_call(n):
    return pl.pallas_call(
        _mlp_body,
        grid=(n // _B,),
        in_specs=[
            pl.BlockSpec((2 * _NUM_LEVELS, _B), lambda i: (0, i)),
            pl.BlockSpec((8, _B), lambda i: (0, i)),
            pl.BlockSpec((64, 32), lambda i: (0, 0)),
            pl.BlockSpec((64, 64), lambda i: (0, 0)),
            pl.BlockSpec((72, 64), lambda i: (0, 0)),
            pl.BlockSpec((64, 80), lambda i: (0, 0)),
            pl.BlockSpec((8, 64), lambda i: (0, 0)),
        ],
        out_specs=pl.BlockSpec((8, _B), lambda i: (0, i)),
        out_shape=jax.ShapeDtypeStruct((8, n), jnp.float32),
    )


def kernel(x, d, table, sw0, sw1, sw2, cw0, cw1):
    xT = x.T
    tab = jnp.transpose(
        jnp.transpose(table, (0, 2, 1)).reshape(_NUM_LEVELS, _LEVEL_DIM,
                                                _T // 128, 128),
        (0, 2, 1, 3),
    ).reshape(_NUM_LEVELS * _T * _LEVEL_DIM)
    packed = _build_pack_sc()(tab)
    dT8 = jnp.concatenate([d.T, jnp.zeros((5, _N), jnp.float32)], axis=0)
    sw0T = sw0.T
    sw1T = sw1.T
    sw2Tp = jnp.concatenate([sw2[:, 1:], sw2[:, :1]], axis=1).T
    sw2Tp = jnp.concatenate([sw2Tp, jnp.zeros((7, 64), jnp.float32)], axis=0)
    cw0T = cw0.T
    cw1T8 = jnp.concatenate([cw1.T, jnp.zeros((5, 64), jnp.float32)], axis=0)
    splits = (0, 3 * _N // 4, _N)
    encs = []
    for i in range(2):
        lo, hi = splits[i], splits[i + 1]
        encs.append(_build_hash_encode_sc(hi - lo)(
            lax.slice(xT, (0, lo), (3, hi)), packed))
    outs = []
    for i in range(2):
        lo, hi = splits[i], splits[i + 1]
        outs.append(_build_mlp_call(hi - lo)(
            encs[i], lax.slice(dT8, (0, lo), (8, hi)),
            sw0T, sw1T, sw2Tp, cw0T, cw1T8))
    out8 = jnp.concatenate(outs, axis=1)
    sigma = out8[3]
    color = out8[:3].T
    return (sigma, color)
